# Initial kernel scaffold; baseline (speedup 1.0000x reference)
import jax
import jax.numpy as jnp
from jax.experimental import pallas as pl


def kernel(points, params):
    raise NotImplementedError("write your pallas kernel here")


# speedup vs baseline: 3.0804x; 3.0804x over previous
import jax
import jax.numpy as jnp
from jax.experimental import pallas as pl
from jax.experimental.pallas import tpu as pltpu

_K = 16
_N = 256
_EPS = 1e-5
_HI = jax.lax.Precision.HIGHEST


def _dot(a, b):
    return jax.lax.dot_general(a, b, (((1,), (0,)), ((), ())),
                               precision=_HI, preferred_element_type=jnp.float32)


def _sigmoid(x):
    return 1.0 / (1.0 + jnp.exp(-x))


def _body(pts_ref, ptsT_ref, *rest):
    refs = rest[:-2]
    out_ref = rest[-2]
    oh_ref = rest[-1]

    p = pts_ref[0]
    pT = ptsT_ref[0]

    acc = jnp.zeros((_N, _N), jnp.float32)
    x2c = jnp.zeros((_N, 1), jnp.float32)
    x2r = jnp.zeros((1, _N), jnp.float32)
    for c in range(3):
        col = p[:, c:c + 1]
        row = pT[c:c + 1, :]
        acc = acc + col * row
        x2c = x2c + col * col
        x2r = x2r + row * row
    d = x2c + x2r - 2.0 * acc

    lane = jax.lax.broadcasted_iota(jnp.int32, (_N, _N), 1)
    dw = d
    for k in range(_K):
        m = jnp.min(dw, axis=1, keepdims=True)
        cand = jnp.where(dw == m, lane, _N)
        j = jnp.min(cand, axis=1, keepdims=True)
        oh = lane == j
        oh_ref[k * _N:(k + 1) * _N, :] = oh.astype(jnp.float32)
        dw = jnp.where(oh, jnp.inf, dw)

    def lfa(f, i0):
        mg, zg, bg, ms, zs, bs, wfg, wfs, bf = [r[...] for r in refs[i0:i0 + 9]]
        ayg = _dot(p, mg)
        azg = _dot(p, zg) + bg
        ays = _dot(f, ms)
        azs = _dot(f, zs) + bs
        cout = wfg.shape[1]
        acc = jnp.zeros((_N, cout), jnp.float32)
        for k in range(_K):
            oh = oh_ref[k * _N:(k + 1) * _N, :]
            gg = jnp.maximum(_dot(oh, ayg) + azg, 0.0)
            gs = jnp.maximum(_dot(oh, ays) + azs, 0.0)
            h = jnp.maximum(_dot(gg, wfg) + _dot(gs, wfs) + bf, 0.0)
            acc = jnp.maximum(acc, h)
        return acc

    f = lfa(p, 0)
    f = lfa(f, 9)
    f = lfa(f, 18)

    c1T, c2T, p1, p2, w1T, b1, w2T, b2 = [r[...] for r in refs[27:35]]
    cc = jnp.mean(f, axis=0, keepdims=True)
    cw = _sigmoid(_dot(jnp.maximum(_dot(cc, c1T), 0.0), c2T))
    pc = jnp.mean(f, axis=1, keepdims=True)
    pw = _sigmoid(_dot(p2, jnp.maximum(_dot(p1, pc), 0.0)))
    g = f * cw * pw
    h = jnp.maximum(_dot(g, w1T) + b1, 0.0)
    out_ref[0] = _dot(h, w2T) + b2


def _prep_lfa(q, cin):
    sg = q['geom_g'] / jnp.sqrt(1.0 + _EPS)
    ss = q['sem_g'] / jnp.sqrt(1.0 + _EPS)
    sf = q['fuse_g'] / jnp.sqrt(1.0 + _EPS)
    wg = q['geom_w'] * sg[:, None]
    ws = q['sem_w'] * ss[:, None]
    wf = q['fuse_w'] * sf[:, None]
    cout = wf.shape[0]
    return [wg[:, 3:].T, (wg[:, :3] - wg[:, 3:]).T, q['geom_b'][None, :],
            ws[:, cin:].T, (ws[:, :cin] - ws[:, cin:]).T, q['sem_b'][None, :],
            wf[:, :cout].T, wf[:, cout:].T, q['fuse_b'][None, :]]


def kernel(points, params):
    b = points.shape[0]
    ptsT = jnp.transpose(points, (0, 2, 1))
    g = params['gfe']
    dec = params['dec']
    weights = (_prep_lfa(params['lfa1'], 3)
               + _prep_lfa(params['lfa2'], 64)
               + _prep_lfa(params['lfa3'], 128)
               + [g['c1'].T, g['c2'].T, g['p1'], g['p2'],
                  dec['w1'].T, dec['b1'][None, :], dec['w2'].T, dec['b2'][None, :]])

    in_specs = [pl.BlockSpec((1, _N, 3), lambda i: (i, 0, 0)),
                pl.BlockSpec((1, 3, _N), lambda i: (i, 0, 0))]
    for w in weights:
        in_specs.append(pl.BlockSpec(w.shape, lambda i, nd=w.ndim: (0,) * nd))

    return pl.pallas_call(
        _body,
        grid=(b,),
        in_specs=in_specs,
        out_specs=pl.BlockSpec((1, _N, 13), lambda i: (i, 0, 0)),
        out_shape=jax.ShapeDtypeStruct((b, _N, 13), jnp.float32),
        scratch_shapes=[pltpu.VMEM((_K * _N, _N), jnp.float32)],
    )(points, ptsT, *weights)


# speedup vs baseline: 5.1420x; 1.6692x over previous
import jax
import jax.numpy as jnp
from jax.experimental import pallas as pl
from jax.experimental.pallas import tpu as pltpu

_K = 16
_N = 256
_EPS = 1e-5
_HI = jax.lax.Precision.HIGHEST


def _dot(a, b, prec=jax.lax.Precision.HIGHEST):
    return jax.lax.dot_general(a, b, (((1,), (0,)), ((), ())),
                               precision=prec, preferred_element_type=jnp.float32)


def _split(x):
    hi = jax.lax.bitcast_convert_type(
        jax.lax.bitcast_convert_type(x, jnp.uint32) & jnp.uint32(0xFFFF0000),
        jnp.float32)
    return hi, x - hi


def _dot3(a, b_hi, b_hi_bf, b_lo_bf):
    a_hi, a_lo = _split(a)
    return (_dot(a_hi, b_hi, None)
            + (_dot(a_hi.astype(jnp.bfloat16), b_lo_bf, None)
               + _dot(a_lo.astype(jnp.bfloat16), b_hi_bf, None)))


def _body(pts_ref, ptsT_ref, *rest):
    refs = rest[:-3]
    out_ref = rest[-3]
    oh_ref = rest[-2]
    ohb_ref = rest[-1]

    p = pts_ref[0]
    pT = ptsT_ref[0]

    acc = jnp.zeros((_N, _N), jnp.float32)
    x2c = jnp.zeros((_N, 1), jnp.float32)
    x2r = jnp.zeros((1, _N), jnp.float32)
    for c in range(3):
        col = p[:, c:c + 1]
        row = pT[c:c + 1, :]
        acc = acc + col * row
        x2c = x2c + col * col
        x2r = x2r + row * row
    d = x2c + x2r - 2.0 * acc

    lane = jax.lax.broadcasted_iota(jnp.int32, (_N, _N), 1)
    dw = d
    for k in range(_K):
        m = jnp.min(dw, axis=1, keepdims=True)
        cand = jnp.where(dw == m, lane, _N)
        j = jnp.min(cand, axis=1, keepdims=True)
        oh = lane == j
        ohf = jnp.where(oh, 1.0, 0.0)
        oh_ref[k * _N:(k + 1) * _N, :] = ohf
        ohb_ref[k * _N:(k + 1) * _N, :] = ohf.astype(jnp.bfloat16)
        dw = jnp.where(oh, jnp.inf, dw)

    def lfa(f, i0, use2pass=True, use3pass=True):
        (mg, zg, bg, ms, zs, bs,
         wf_hi, wf_hi_bf, wf_lo_bf, bf) = [r[...] for r in refs[i0:i0 + 10]]
        ayg = _dot(p, mg)
        ays = _dot(f, ms)
        az = jnp.concatenate([_dot(p, zg) + bg, _dot(f, zs) + bs], axis=1)
        ayg_hi, ayg_lo = _split(ayg)
        ays_hi, ays_lo = _split(ays)
        ay_hi = jnp.concatenate([ayg_hi, ays_hi], axis=1)
        ay_lo_bf = jnp.concatenate([ayg_lo, ays_lo], axis=1).astype(jnp.bfloat16)
        cout = wf_hi.shape[1]
        acc = jnp.zeros((_N, cout), jnp.float32)
        for k in range(_K):
            oh = oh_ref[k * _N:(k + 1) * _N, :]
            ohb = ohb_ref[k * _N:(k + 1) * _N, :]
            if use2pass:
                gsel = _dot(oh, ay_hi, None) + _dot(ohb, ay_lo_bf, None)
            else:
                gsel = _dot(oh, ay_hi + (ay_lo_bf.astype(jnp.float32)))
            g = jnp.maximum(gsel + az, 0.0)
            if use3pass:
                h = _dot3(g, wf_hi, wf_hi_bf, wf_lo_bf)
            else:
                h = _dot(g, wf_hi + wf_lo_bf.astype(jnp.float32))
            h = jnp.maximum(h + bf, 0.0)
            acc = jnp.maximum(acc, h)
        return acc

    f = lfa(p, 0, use2pass=False, use3pass=False)
    f = lfa(f, 10, use2pass=False, use3pass=True)
    f = lfa(f, 20, use2pass=True, use3pass=True)

    (c1T, c2T, p1, p2,
     w1_hi, w1_hi_bf, w1_lo_bf, b1, w2T, b2) = [r[...] for r in refs[30:40]]
    cc = jnp.mean(f, axis=0, keepdims=True)
    cw = _sigmoid(_dot(jnp.maximum(_dot(cc, c1T), 0.0), c2T))
    pc = jnp.mean(f, axis=1, keepdims=True)
    pw = _sigmoid(_dot(p2, jnp.maximum(_dot(p1, pc), 0.0)))
    g = f * cw * pw
    h = jnp.maximum(_dot(g, w1_hi + w1_lo_bf.astype(jnp.float32)) + b1, 0.0)
    out_ref[0] = _dot(h, w2T) + b2


def _sigmoid(x):
    return 1.0 / (1.0 + jnp.exp(-x))


def _prep_lfa(q, cin):
    sg = q['geom_g'] / jnp.sqrt(1.0 + _EPS)
    ss = q['sem_g'] / jnp.sqrt(1.0 + _EPS)
    sf = q['fuse_g'] / jnp.sqrt(1.0 + _EPS)
    wg = q['geom_w'] * sg[:, None]
    ws = q['sem_w'] * ss[:, None]
    wf = q['fuse_w'] * sf[:, None]
    return [wg[:, 3:].T, (wg[:, :3] - wg[:, 3:]).T, q['geom_b'][None, :],
            ws[:, cin:].T, (ws[:, :cin] - ws[:, cin:]).T, q['sem_b'][None, :],
            *_split3(wf.T), q['fuse_b'][None, :]]


def _split3(w):
    w_hi, w_lo = _split(w)
    return [w_hi, w_hi.astype(jnp.bfloat16), w_lo.astype(jnp.bfloat16)]


def kernel(points, params):
    b = points.shape[0]
    ptsT = jnp.transpose(points, (0, 2, 1))
    g = params['gfe']
    dec = params['dec']
    weights = (_prep_lfa(params['lfa1'], 3)
               + _prep_lfa(params['lfa2'], 64)
               + _prep_lfa(params['lfa3'], 128)
               + [g['c1'].T, g['c2'].T, g['p1'], g['p2'],
                  *_split3(dec['w1'].T), dec['b1'][None, :],
                  dec['w2'].T, dec['b2'][None, :]])

    in_specs = [pl.BlockSpec((1, _N, 3), lambda i: (i, 0, 0)),
                pl.BlockSpec((1, 3, _N), lambda i: (i, 0, 0))]
    for w in weights:
        in_specs.append(pl.BlockSpec(w.shape, lambda i, nd=w.ndim: (0,) * nd))

    return pl.pallas_call(
        _body,
        grid=(b,),
        in_specs=in_specs,
        out_specs=pl.BlockSpec((1, _N, 13), lambda i: (i, 0, 0)),
        out_shape=jax.ShapeDtypeStruct((b, _N, 13), jnp.float32),
        scratch_shapes=[pltpu.VMEM((_K * _N, _N), jnp.float32),
                        pltpu.VMEM((_K * _N, _N), jnp.bfloat16)],
    )(points, ptsT, *weights)


# speedup vs baseline: 5.4132x; 1.0527x over previous
import jax
import jax.numpy as jnp
from jax.experimental import pallas as pl
from jax.experimental.pallas import tpu as pltpu

_K = 16
_N = 256
_EPS = 1e-5
_HI = jax.lax.Precision.HIGHEST


def _dot(a, b, prec=jax.lax.Precision.HIGHEST):
    return jax.lax.dot_general(a, b, (((1,), (0,)), ((), ())),
                               precision=prec, preferred_element_type=jnp.float32)


def _split(x):
    hi = jax.lax.bitcast_convert_type(
        jax.lax.bitcast_convert_type(x, jnp.uint32) & jnp.uint32(0xFFFF0000),
        jnp.float32)
    return hi, x - hi


def _dot3(a, b_hi, b_hi_bf, b_lo_bf):
    a_hi, a_lo = _split(a)
    return (_dot(a_hi, b_hi, None)
            + (_dot(a_hi.astype(jnp.bfloat16), b_lo_bf, None)
               + _dot(a_lo.astype(jnp.bfloat16), b_hi_bf, None)))


def _body(pts_ref, ptsT_ref, *rest):
    refs = rest[:-3]
    out_ref = rest[-3]
    oh_ref = rest[-2]
    ohb_ref = rest[-1]

    p = pts_ref[0]
    pT = ptsT_ref[0]

    acc = jnp.zeros((_N, _N), jnp.float32)
    x2c = jnp.zeros((_N, 1), jnp.float32)
    x2r = jnp.zeros((1, _N), jnp.float32)
    for c in range(3):
        col = p[:, c:c + 1]
        row = pT[c:c + 1, :]
        acc = acc + col * row
        x2c = x2c + col * col
        x2r = x2r + row * row
    d = x2c + x2r - 2.0 * acc

    lane = jax.lax.broadcasted_iota(jnp.int32, (_N, _N), 1)
    dw = d
    for k in range(_K):
        m = jnp.min(dw, axis=1, keepdims=True)
        cand = jnp.where(dw == m, lane, _N)
        j = jnp.min(cand, axis=1, keepdims=True)
        oh = lane == j
        ohf = jnp.where(oh, 1.0, 0.0)
        oh_ref[k * _N:(k + 1) * _N, :] = ohf
        ohb_ref[k * _N:(k + 1) * _N, :] = ohf.astype(jnp.bfloat16)
        dw = jnp.where(oh, jnp.inf, dw)

    def lfa(f, i0, use2pass=True, use3pass=True):
        (mg, zg, bg, ms, zs, bs,
         wf_hi, wf_hi_bf, wf_lo_bf, bf) = [r[...] for r in refs[i0:i0 + 10]]
        ayg = _dot(p, mg)
        ays = _dot(f, ms)
        az = jnp.concatenate([_dot(p, zg) + bg, _dot(f, zs) + bs], axis=1)
        ayg_hi, ayg_lo = _split(ayg)
        ays_hi, ays_lo = _split(ays)
        ay_hi = jnp.concatenate([ayg_hi, ays_hi], axis=1)
        ay_lo_bf = jnp.concatenate([ayg_lo, ays_lo], axis=1).astype(jnp.bfloat16)
        cout = wf_hi.shape[1]
        acc = jnp.zeros((_N, cout), jnp.float32)
        for k in range(_K):
            oh = oh_ref[k * _N:(k + 1) * _N, :]
            ohb = ohb_ref[k * _N:(k + 1) * _N, :]
            if use2pass:
                gsel = _dot(oh, ay_hi, None) + _dot(ohb, ay_lo_bf, None)
            else:
                gsel = _dot(oh, ay_hi + (ay_lo_bf.astype(jnp.float32)))
            g = jnp.maximum(gsel + az, 0.0)
            if use3pass:
                h = _dot3(g, wf_hi, wf_hi_bf, wf_lo_bf)
            else:
                h = _dot(g, wf_hi + wf_lo_bf.astype(jnp.float32))
            h = jnp.maximum(h + bf, 0.0)
            acc = jnp.maximum(acc, h)
        return acc

    f = lfa(p, 0, use2pass=False, use3pass=True)
    f = lfa(f, 10, use2pass=False, use3pass=True)
    f = lfa(f, 20, use2pass=True, use3pass=True)

    (c1T, c2T, p1, p2,
     w1_hi, w1_hi_bf, w1_lo_bf, b1, w2T, b2) = [r[...] for r in refs[30:40]]
    cc = jnp.mean(f, axis=0, keepdims=True)
    cw = _sigmoid(_dot(jnp.maximum(_dot(cc, c1T), 0.0), c2T))
    pc = jnp.mean(f, axis=1, keepdims=True)
    pw = _sigmoid(_dot(p2, jnp.maximum(_dot(p1, pc), 0.0)))
    g = f * cw * pw
    h = jnp.maximum(_dot3(g, w1_hi, w1_hi_bf, w1_lo_bf) + b1, 0.0)
    out_ref[0] = _dot(h, w2T) + b2


def _sigmoid(x):
    return 1.0 / (1.0 + jnp.exp(-x))


def _prep_lfa(q, cin):
    sg = q['geom_g'] / jnp.sqrt(1.0 + _EPS)
    ss = q['sem_g'] / jnp.sqrt(1.0 + _EPS)
    sf = q['fuse_g'] / jnp.sqrt(1.0 + _EPS)
    wg = q['geom_w'] * sg[:, None]
    ws = q['sem_w'] * ss[:, None]
    wf = q['fuse_w'] * sf[:, None]
    return [wg[:, 3:].T, (wg[:, :3] - wg[:, 3:]).T, q['geom_b'][None, :],
            ws[:, cin:].T, (ws[:, :cin] - ws[:, cin:]).T, q['sem_b'][None, :],
            *_split3(wf.T), q['fuse_b'][None, :]]


def _split3(w):
    w_hi, w_lo = _split(w)
    return [w_hi, w_hi.astype(jnp.bfloat16), w_lo.astype(jnp.bfloat16)]


def kernel(points, params):
    b = points.shape[0]
    ptsT = jnp.transpose(points, (0, 2, 1))
    g = params['gfe']
    dec = params['dec']
    weights = (_prep_lfa(params['lfa1'], 3)
               + _prep_lfa(params['lfa2'], 64)
               + _prep_lfa(params['lfa3'], 128)
               + [g['c1'].T, g['c2'].T, g['p1'], g['p2'],
                  *_split3(dec['w1'].T), dec['b1'][None, :],
                  dec['w2'].T, dec['b2'][None, :]])

    in_specs = [pl.BlockSpec((1, _N, 3), lambda i: (i, 0, 0)),
                pl.BlockSpec((1, 3, _N), lambda i: (i, 0, 0))]
    for w in weights:
        in_specs.append(pl.BlockSpec(w.shape, lambda i, nd=w.ndim: (0,) * nd))

    return pl.pallas_call(
        _body,
        grid=(b,),
        in_specs=in_specs,
        out_specs=pl.BlockSpec((1, _N, 13), lambda i: (i, 0, 0)),
        out_shape=jax.ShapeDtypeStruct((b, _N, 13), jnp.float32),
        scratch_shapes=[pltpu.VMEM((_K * _N, _N), jnp.float32),
                        pltpu.VMEM((_K * _N, _N), jnp.bfloat16)],
    )(points, ptsT, *weights)
